# baseline (speedup 1.0000x reference)
import functools

import jax
import jax.numpy as jnp
from jax.experimental import pallas as pl
from jax.experimental.pallas import tpu as pltpu

_VMEM_LIMIT = 64 * 1024 * 1024


def _se_fused_kernel(x_ref, w1t_ref, w2t_ref, o_ref, *, inv_hw):
    x = x_ref[...]
    p = jnp.sum(x, axis=(2, 3), dtype=jnp.float32) * inv_hw
    h = jnp.dot(p, w1t_ref[...].astype(jnp.float32),
                preferred_element_type=jnp.float32)
    h = jnp.maximum(h, 0.0)
    g = jnp.dot(h, w2t_ref[...].astype(jnp.float32),
                preferred_element_type=jnp.float32)
    g = jax.nn.sigmoid(g)
    o_ref[...] = x * g[:, :, None, None].astype(o_ref.dtype)


def kernel(x, w1, w2):
    B, C, H, W = x.shape
    HW = H * W
    C_red = w1.shape[0]

    itemsize = jnp.dtype(x.dtype).itemsize
    bb = 2 if B % 2 == 0 else 1

    w1t = jnp.transpose(w1)
    w2t = jnp.transpose(w2)

    return pl.pallas_call(
        functools.partial(_se_fused_kernel, inv_hw=1.0 / HW),
        out_shape=jax.ShapeDtypeStruct((B, C, H, W), x.dtype),
        grid=(B // bb,),
        in_specs=[
            pl.BlockSpec((bb, C, H, W), lambda b: (b, 0, 0, 0)),
            pl.BlockSpec((C, C_red), lambda b: (0, 0)),
            pl.BlockSpec((C_red, C), lambda b: (0, 0)),
        ],
        out_specs=pl.BlockSpec((bb, C, H, W), lambda b: (b, 0, 0, 0)),
        compiler_params=pltpu.CompilerParams(
            dimension_semantics=("parallel",),
            vmem_limit_bytes=_VMEM_LIMIT),
        cost_estimate=pl.CostEstimate(
            flops=2 * B * C * HW + 4 * B * C * C_red,
            transcendentals=B * C,
            bytes_accessed=2 * B * C * HW * itemsize + 2 * C * C_red * 4),
    )(x, w1t, w2t)


# speedup vs baseline: 12.8770x; 12.8770x over previous
import functools

import jax
import jax.numpy as jnp
from jax.experimental import pallas as pl
from jax.experimental.pallas import tpu as pltpu

_VMEM_LIMIT = 64 * 1024 * 1024


def _se_fused_kernel(x_ref, w1t_ref, w2t_ref, o_ref, *, inv_hw):
    x = x_ref[...]
    p = jnp.sum(x, axis=(1, 2), dtype=jnp.float32) * inv_hw
    h = jnp.dot(p, w1t_ref[...].astype(jnp.float32),
                preferred_element_type=jnp.float32)
    h = jnp.maximum(h, 0.0)
    g = jnp.dot(h, w2t_ref[...].astype(jnp.float32),
                preferred_element_type=jnp.float32)
    g = jax.nn.sigmoid(g)
    o_ref[...] = x * g[:, None, None, :].astype(o_ref.dtype)


def kernel(x, w1, w2):
    B, C, H, W = x.shape
    HW = H * W
    C_red = w1.shape[0]
    itemsize = jnp.dtype(x.dtype).itemsize

    x_nhwc = jnp.transpose(x, (0, 2, 3, 1))
    w1t = jnp.transpose(w1)
    w2t = jnp.transpose(w2)

    bb = 1
    for cand in (8, 4, 2):
        if B % cand == 0 and cand * C * HW * itemsize * 4 <= 48 * 1024 * 1024:
            bb = cand
            break

    out_nhwc = pl.pallas_call(
        functools.partial(_se_fused_kernel, inv_hw=1.0 / HW),
        out_shape=jax.ShapeDtypeStruct((B, H, W, C), x.dtype),
        grid=(B // bb,),
        in_specs=[
            pl.BlockSpec((bb, H, W, C), lambda b: (b, 0, 0, 0)),
            pl.BlockSpec((C, C_red), lambda b: (0, 0)),
            pl.BlockSpec((C_red, C), lambda b: (0, 0)),
        ],
        out_specs=pl.BlockSpec((bb, H, W, C), lambda b: (b, 0, 0, 0)),
        compiler_params=pltpu.CompilerParams(
            dimension_semantics=("parallel",),
            vmem_limit_bytes=_VMEM_LIMIT),
        cost_estimate=pl.CostEstimate(
            flops=2 * B * C * HW + 4 * B * C * C_red,
            transcendentals=B * C,
            bytes_accessed=2 * B * C * HW * itemsize + 2 * C * C_red * 4),
    )(x_nhwc, w1t, w2t)

    return jnp.transpose(out_nhwc, (0, 3, 1, 2))
